# baseline (speedup 1.0000x reference)
import functools

import jax
import jax.numpy as jnp
from jax import lax
from jax.experimental import pallas as pl
from jax.experimental.pallas import tpu as pltpu
from jax.experimental.pallas import tpu_sc as plsc

N_BBLK = 8
N_LGRP = 4
LANE = 128
VEC = 16


def kernel(idx, targets, token_embedding_table):
    del targets
    B, L = idx.shape
    V, D = token_embedding_table.shape
    n_strip = (D + LANE - 1) // LANE
    n_full = D // LANE
    rem = D - n_full * LANE

    tabs = (
        jnp.pad(token_embedding_table, ((0, 0), (0, n_strip * LANE - D)))
        .reshape(V, n_strip, LANE)
        .transpose(1, 0, 2)
    )
    idx_t = idx.astype(jnp.int32).T.reshape(L, N_BBLK, LANE)

    mesh = plsc.VectorSubcoreMesh(core_axis_name="c", subcore_axis_name="s")

    @functools.partial(
        pl.kernel,
        out_type=jax.ShapeDtypeStruct((L, D, B), jnp.float32),
        mesh=mesh,
        compiler_params=pltpu.CompilerParams(needs_layout_passes=False),
        scratch_types=[
            pltpu.VMEM((LANE,), jnp.int32),
            pltpu.VMEM((LANE, LANE), jnp.float32),
            pltpu.VMEM((LANE, LANE), jnp.float32),
            pltpu.SemaphoreType.DMA,
            pltpu.SemaphoreType.DMA,
        ],
    )
    def gather_kernel(tabs_hbm, idx_hbm, out_hbm, idx_v, buf_in, buf_out, gs, ws):
        wid = lax.axis_index("s") * 2 + lax.axis_index("c")
        bblk = wid % N_BBLK
        lg = wid // N_BBLK

        rows = [lax.iota(jnp.int32, VEC) + k * VEC for k in range(LANE // VEC)]

        def transpose_tile():
            @pl.loop(0, LANE, unroll=8)
            def _per_d(d):
                col = jnp.full((VEC,), d, jnp.int32)
                for k in range(LANE // VEC):
                    buf_out[d, pl.ds(k * VEC, VEC)] = plsc.load_gather(
                        buf_in, [rows[k], col]
                    )

        for c in range(n_strip):
            d_lo = c * LANE
            d_sz = LANE if c < n_full else rem

            @pl.loop(lg, L, step=N_LGRP)
            def _per_l(l):
                pltpu.sync_copy(idx_hbm.at[l, bblk], idx_v)
                pltpu.async_copy(tabs_hbm.at[c].at[idx_v], buf_in, gs).wait()
                transpose_tile()
                pltpu.async_copy(
                    buf_out.at[pl.ds(0, d_sz)],
                    out_hbm.at[l, pl.ds(d_lo, d_sz), pl.ds(bblk * LANE, LANE)],
                    ws,
                ).wait()

    out = gather_kernel(tabs, idx_t)
    return out.transpose(2, 0, 1)


# speedup vs baseline: 3.8294x; 3.8294x over previous
import functools

import jax
import jax.numpy as jnp
from jax import lax
from jax.experimental import pallas as pl
from jax.experimental.pallas import tpu as pltpu
from jax.experimental.pallas import tpu_sc as plsc

NUM_WORKERS = 32
LANE = 128
VEC = 16


def kernel(idx, targets, token_embedding_table):
    del targets
    B, L = idx.shape
    V, D = token_embedding_table.shape
    n_strip = (D + LANE - 1) // LANE
    n_full = D // LANE
    rem = D - n_full * LANE
    n_ch = B // NUM_WORKERS

    tabs = (
        jnp.pad(token_embedding_table, ((0, 0), (0, n_strip * LANE - D)))
        .reshape(V, n_strip, LANE)
        .transpose(1, 0, 2)
    )
    idx_i32 = idx.astype(jnp.int32)

    mesh = plsc.VectorSubcoreMesh(core_axis_name="c", subcore_axis_name="s")

    @functools.partial(
        pl.kernel,
        out_type=jax.ShapeDtypeStruct((B, L, D), jnp.float32),
        mesh=mesh,
        compiler_params=pltpu.CompilerParams(needs_layout_passes=False),
        scratch_types=[
            pltpu.VMEM((n_ch, L), jnp.int32),
            pltpu.VMEM((L, D), jnp.float32),
            pltpu.VMEM((L, D), jnp.float32),
            pltpu.VMEM((L, LANE), jnp.float32),
            pltpu.SemaphoreType.DMA,
            pltpu.SemaphoreType.DMA,
            pltpu.SemaphoreType.DMA,
            pltpu.SemaphoreType.DMA,
        ],
    )
    def gather_kernel(
        tabs_hbm, idx_hbm, out_hbm, idx_v, buf0, buf1, tail_v, g0, g1, w0, w1
    ):
        wid = lax.axis_index("s") * 2 + lax.axis_index("c")
        base_b = wid * n_ch
        pltpu.sync_copy(idx_hbm.at[pl.ds(base_b, n_ch)], idx_v)

        def gather_all(j, buf, sem):
            for c in range(n_full):
                pltpu.async_copy(
                    tabs_hbm.at[c].at[idx_v.at[j]],
                    buf.at[:, pl.ds(c * LANE, LANE)],
                    sem,
                )
            pltpu.async_copy(tabs_hbm.at[n_full].at[idx_v.at[j]], tail_v, sem)

        def wait_gather_all(j, buf, sem):
            for c in range(n_full):
                pltpu.make_async_copy(
                    tabs_hbm.at[c].at[idx_v.at[j]],
                    buf.at[:, pl.ds(c * LANE, LANE)],
                    sem,
                ).wait()
            pltpu.make_async_copy(
                tabs_hbm.at[n_full].at[idx_v.at[j]], tail_v, sem
            ).wait()

        def patch_tail(buf):
            n_vec = rem // VEC
            tail8 = rem - n_vec * VEC
            lane = lax.iota(jnp.int32, VEC)
            for r in range(L):
                for k in range(n_vec):
                    buf[r, pl.ds(n_full * LANE + k * VEC, VEC)] = tail_v[
                        r, pl.ds(k * VEC, VEC)
                    ]
                val = tail_v[r, pl.ds(n_vec * VEC, VEC)]
                plsc.store_scatter(
                    buf,
                    [
                        jnp.full((VEC,), r, jnp.int32),
                        n_full * LANE + n_vec * VEC + lane,
                    ],
                    val,
                    mask=lane < tail8,
                )

        def write(j, buf, sem):
            return pltpu.async_copy(buf, out_hbm.at[base_b + j], sem)

        def wait_write(j, buf, sem):
            pltpu.make_async_copy(buf, out_hbm.at[base_b + j], sem).wait()

        gather_all(0, buf0, g0)

        @pl.loop(0, n_ch, step=2)
        def _pair(j):
            wait_gather_all(j, buf0, g0)
            patch_tail(buf0)

            @pl.when(j > 0)
            def _():
                wait_write(j - 1, buf1, w1)

            gather_all(j + 1, buf1, g1)
            write(j, buf0, w0)

            wait_gather_all(j + 1, buf1, g1)
            patch_tail(buf1)

            @pl.when(j + 2 < n_ch)
            def _():
                wait_write(j, buf0, w0)
                gather_all(j + 2, buf0, g0)

            write(j + 1, buf1, w1)

        wait_write(n_ch - 2, buf0, w0)
        wait_write(n_ch - 1, buf1, w1)

    return gather_kernel(tabs, idx_i32)


# speedup vs baseline: 3.8676x; 1.0100x over previous
import functools

import jax
import jax.numpy as jnp
from jax import lax
from jax.experimental import pallas as pl
from jax.experimental.pallas import tpu as pltpu
from jax.experimental.pallas import tpu_sc as plsc

NUM_WORKERS = 32
LANE = 128
VEC = 16


def kernel(idx, targets, token_embedding_table):
    del targets
    B, L = idx.shape
    V, D = token_embedding_table.shape
    n_strip = (D + LANE - 1) // LANE
    n_full = D // LANE
    rem = D - n_full * LANE
    n_ch = B // NUM_WORKERS

    tabs = (
        jnp.pad(token_embedding_table, ((0, 0), (0, n_strip * LANE - D)))
        .reshape(V, n_strip, LANE)
        .transpose(1, 0, 2)
    )
    idx_i32 = idx.astype(jnp.int32)

    mesh = plsc.VectorSubcoreMesh(core_axis_name="c", subcore_axis_name="s")

    @functools.partial(
        pl.kernel,
        out_type=jax.ShapeDtypeStruct((B, L, D), jnp.float32),
        mesh=mesh,
        compiler_params=pltpu.CompilerParams(needs_layout_passes=False),
        scratch_types=[
            pltpu.VMEM((n_ch, L), jnp.int32),
            pltpu.VMEM((L, D), jnp.float32),
            pltpu.VMEM((L, D), jnp.float32),
            pltpu.VMEM((L, LANE), jnp.float32),
            pltpu.SemaphoreType.DMA,
            pltpu.SemaphoreType.DMA,
            pltpu.SemaphoreType.DMA,
            pltpu.SemaphoreType.DMA,
            pltpu.SemaphoreType.DMA,
            pltpu.SemaphoreType.DMA,
        ],
    )
    def gather_kernel(
        tabs_hbm, idx_hbm, out_hbm, idx_v, buf0, buf1, tail_v,
        g0, g1, w0, w1, t0, t1,
    ):
        wid = lax.axis_index("s") * 2 + lax.axis_index("c")
        base_b = wid * n_ch
        pltpu.sync_copy(idx_hbm.at[pl.ds(base_b, n_ch)], idx_v)

        def gather_strips(j, buf, sem):
            for c in range(n_full):
                pltpu.async_copy(
                    tabs_hbm.at[c].at[idx_v.at[j]],
                    buf.at[:, pl.ds(c * LANE, LANE)],
                    sem,
                )

        def wait_strips(j, buf, sem):
            for c in range(n_full):
                pltpu.make_async_copy(
                    tabs_hbm.at[c].at[idx_v.at[j]],
                    buf.at[:, pl.ds(c * LANE, LANE)],
                    sem,
                ).wait()

        def gather_tail(j, sem):
            pltpu.async_copy(tabs_hbm.at[n_full].at[idx_v.at[j]], tail_v, sem)

        def wait_tail(j, sem):
            pltpu.make_async_copy(
                tabs_hbm.at[n_full].at[idx_v.at[j]], tail_v, sem
            ).wait()

        def patch_tail(buf):
            n_vec = rem // VEC
            tail8 = rem - n_vec * VEC
            lane = lax.iota(jnp.int32, VEC)
            for r in range(L):
                for k in range(n_vec):
                    buf[r, pl.ds(n_full * LANE + k * VEC, VEC)] = tail_v[
                        r, pl.ds(k * VEC, VEC)
                    ]
                val = tail_v[r, pl.ds(n_vec * VEC, VEC)]
                plsc.store_scatter(
                    buf,
                    [
                        jnp.full((VEC,), r, jnp.int32),
                        n_full * LANE + n_vec * VEC + lane,
                    ],
                    val,
                    mask=lane < tail8,
                )

        def write(j, buf, sem):
            return pltpu.async_copy(buf, out_hbm.at[base_b + j], sem)

        def wait_write(j, buf, sem):
            pltpu.make_async_copy(buf, out_hbm.at[base_b + j], sem).wait()

        gather_strips(0, buf0, g0)
        gather_tail(0, t0)

        @pl.loop(0, n_ch, step=2)
        def _pair(j):
            @pl.when(j > 0)
            def _():
                wait_write(j - 1, buf1, w1)

            gather_strips(j + 1, buf1, g1)
            wait_tail(j, t0)
            patch_tail(buf0)
            gather_tail(j + 1, t1)
            wait_strips(j, buf0, g0)
            write(j, buf0, w0)

            @pl.when(j + 2 < n_ch)
            def _():
                wait_write(j, buf0, w0)
                gather_strips(j + 2, buf0, g0)

            wait_tail(j + 1, t1)
            patch_tail(buf1)

            @pl.when(j + 2 < n_ch)
            def _():
                gather_tail(j + 2, t0)

            wait_strips(j + 1, buf1, g1)
            write(j + 1, buf1, w1)

        wait_write(n_ch - 2, buf0, w0)
        wait_write(n_ch - 1, buf1, w1)

    return gather_kernel(tabs, idx_i32)
